# baseline (speedup 1.0000x reference)
import functools

import jax
import jax.numpy as jnp
import numpy as np
from jax import lax
from jax.experimental import pallas as pl
from jax.experimental.pallas import tpu as pltpu
from jax.experimental.pallas import tpu_sc as plsc

B = 4096
L = 200
D = 64
NUM_CORES = 2
NUM_SUBCORES = 16
NW = NUM_CORES * NUM_SUBCORES
N = B * L
PER_W = N // NW
C = 800
NWIN = PER_W // C
QQ = C // L


def _pe_table() -> jnp.ndarray:
    pos = np.arange(L, dtype=np.float32)[:, None]
    i = np.arange(0, D, 2, dtype=np.float32)
    div = np.exp(-np.log(10000.0) * i / float(D))
    pe = np.zeros((L, D), dtype=np.float32)
    pe[:, 0::2] = np.sin(pos * div)
    pe[:, 1::2] = np.cos(pos * div)
    return jnp.asarray(pe)


_mesh = plsc.VectorSubcoreMesh(
    core_axis_name="c", subcore_axis_name="s",
    num_cores=NUM_CORES, num_subcores=NUM_SUBCORES)


@functools.partial(
    pl.kernel,
    out_type=jax.ShapeDtypeStruct((N, D), jnp.float32),
    mesh=_mesh,
    scratch_types=[
        pltpu.VMEM((2, C), jnp.int32),
        pltpu.VMEM((2, C, D), jnp.float32),
        pltpu.VMEM((L, D), jnp.float32),
        [pltpu.SemaphoreType.DMA] * 2,
        [pltpu.SemaphoreType.DMA] * 2,
        [pltpu.SemaphoreType.DMA] * 2,
    ],
    compiler_params=pltpu.CompilerParams(use_tc_tiling_on_sc=False),
)
def _emb_sc(tgt_hbm, pe_hbm, table_hbm, out_hbm,
            idx_v, rows_v, pe_v, isem, gsem, osem):
    wid = lax.axis_index("s") * NUM_CORES + lax.axis_index("c")
    base = wid * PER_W
    pltpu.sync_copy(pe_hbm, pe_v)

    def idx_copy(w, b):
        return pltpu.make_async_copy(
            tgt_hbm.at[pl.ds(base + w * C, C)], idx_v.at[b], isem[b])

    def gather(b):
        return pltpu.make_async_copy(
            table_hbm.at[idx_v.at[b]], rows_v.at[b], gsem[b])

    def out_copy(w, b):
        return pltpu.make_async_copy(
            rows_v.at[b], out_hbm.at[pl.ds(base + w * C, C)], osem[b])

    idx_copy(0, 0).start()
    idx_copy(1, 1).start()
    idx_copy(0, 0).wait()
    gather(0).start()

    def step(w, b):
        @pl.when((w >= 1) & (w + 1 < NWIN))
        def _():
            out_copy(w - 1, 1 - b).wait()

        @pl.when(w + 1 < NWIN)
        def _():
            idx_copy(w + 1, 1 - b).wait()
            gather(1 - b).start()

        gather(b).wait()

        @pl.when(w + 2 < NWIN)
        def _():
            idx_copy(w + 2, b).start()

        def fuse(l, _):
            for q in range(QQ):
                r = q * L + l
                for j in range(D // 16):
                    sl = pl.ds(j * 16, 16)
                    rows_v[b, r, sl] = rows_v[b, r, sl] * 8.0 + pe_v[l, sl]
            return 0

        lax.fori_loop(0, L, fuse, 0, unroll=2)
        out_copy(w, b).start()

    def outer(g, _):
        for b in range(2):
            step(g * 2 + b, b)
        return 0

    lax.fori_loop(0, NWIN // 2, outer, 0)

    out_copy(NWIN - 2, 0).wait()
    out_copy(NWIN - 1, 1).wait()


def _mask_body(o_ref):
    r = lax.broadcasted_iota(jnp.int32, (L, L), 0)
    c = lax.broadcasted_iota(jnp.int32, (L, L), 1)
    o_ref[...] = jnp.where(r >= c, jnp.float32(0.0), jnp.float32(-jnp.inf))


_mask_call = pl.pallas_call(
    _mask_body,
    out_shape=jax.ShapeDtypeStruct((L, L), jnp.float32),
)


def kernel(tgt, table):
    tgt = tgt.astype(jnp.int32).reshape(N)
    emb = _emb_sc(tgt, _pe_table(), table).reshape(B, L, D)
    return emb, _mask_call()
